# baseline (speedup 1.0000x reference)
import functools

import jax
import jax.numpy as jnp
from jax import lax
from jax.experimental import pallas as pl
from jax.experimental.pallas import tpu as pltpu
from jax.experimental.pallas import tpu_sc as plsc


def kernel(x, pos_emb_table):
    B, S, D = x.shape
    NW = 32
    ROWS = B * S
    rows_per_w = ROWS // NW
    R = 8
    CH = rows_per_w // R
    CHUNK = R * D

    mesh = plsc.VectorSubcoreMesh(core_axis_name="c", subcore_axis_name="s")

    @functools.partial(
        pl.kernel,
        out_type=jax.ShapeDtypeStruct((B * S * D,), jnp.float32),
        mesh=mesh,
        scratch_types=[
            pltpu.VMEM((CHUNK,), jnp.float32),
            pltpu.VMEM((CHUNK,), jnp.float32),
        ],
    )
    def sc_add(x_hbm, pos_hbm, out_hbm, xv, pv):
        wid = lax.axis_index("s") * 2 + lax.axis_index("c")
        row0 = wid * rows_per_w

        def chunk_body(c, carry):
            row = row0 + c * R
            xbase = row * D
            pbase = (row % S) * D
            pltpu.sync_copy(x_hbm.at[pl.ds(xbase, CHUNK)], xv)
            pltpu.sync_copy(pos_hbm.at[pl.ds(pbase, CHUNK)], pv)

            @plsc.parallel_loop(0, CHUNK, 16, unroll=8)
            def _(off):
                xv[pl.ds(off, 16)] = xv[pl.ds(off, 16)] + pv[pl.ds(off, 16)]
            pltpu.sync_copy(xv, out_hbm.at[pl.ds(xbase, CHUNK)])
            return carry

        lax.fori_loop(0, CH, chunk_body, 0)

    out = sc_add(x.reshape(B * S * D), pos_emb_table.reshape(S * D))
    return out.reshape(B, S, D)


# speedup vs baseline: 5.3482x; 5.3482x over previous
import jax
import jax.numpy as jnp
from jax.experimental import pallas as pl


def _add_block(x_ref, pos_ref, o_ref):
    o_ref[...] = x_ref[...] + pos_ref[...]


def kernel(x, pos_emb_table):
    B, S, D = x.shape
    BS = 512
    BD = D // 2
    return pl.pallas_call(
        _add_block,
        grid=(S // BS, 2, B),
        in_specs=[
            pl.BlockSpec((1, BS, BD), lambda s, d, b: (b, s, d)),
            pl.BlockSpec((BS, BD), lambda s, d, b: (s, d)),
        ],
        out_specs=pl.BlockSpec((1, BS, BD), lambda s, d, b: (b, s, d)),
        out_shape=jax.ShapeDtypeStruct(x.shape, x.dtype),
    )(x, pos_emb_table)


# speedup vs baseline: 5.5417x; 1.0362x over previous
import jax
import jax.numpy as jnp
from jax.experimental import pallas as pl


def _add_block(x_ref, pos_ref, o_ref):
    o_ref[...] = x_ref[...] + pos_ref[...]


def kernel(x, pos_emb_table):
    B, S, D = x.shape
    BS = 512
    return pl.pallas_call(
        _add_block,
        grid=(S // BS, B),
        in_specs=[
            pl.BlockSpec((1, BS, D), lambda s, b: (b, s, 0)),
            pl.BlockSpec((BS, D), lambda s, b: (s, 0)),
        ],
        out_specs=pl.BlockSpec((1, BS, D), lambda s, b: (b, s, 0)),
        out_shape=jax.ShapeDtypeStruct(x.shape, x.dtype),
    )(x, pos_emb_table)
